# baseline (speedup 1.0000x reference)
import functools

import jax
import jax.numpy as jnp
from jax import lax
from jax.experimental import pallas as pl
from jax.experimental.pallas import tpu as pltpu
from jax.experimental.pallas import tpu_sc as plsc

_B, _T, _D = 16, 576, 256
_K = 1024
_N = _B * _T
_BN = 1024

_NC, _NS = 2, 16
_NW = _NC * _NS
_BPW = _N // _NW


def _tc_body(x_ref, e_hbm, ind_ref, dist_ref, e_v, et_v, e2_v, sem):
    i = pl.program_id(0)

    @pl.when(i == 0)
    def _init():
        cp = pltpu.make_async_copy(e_hbm, e_v, sem)
        cp.start()
        cp.wait()
        et = jnp.swapaxes(e_v[...], 0, 1)
        et_v[...] = et
        e2_v[...] = jnp.sum(et * et, axis=0, keepdims=True)

    x = x_ref[...]
    et = et_v[...]
    xe = jax.lax.dot_general(
        x, et, (((1,), (0,)), ((), ())),
        preferred_element_type=jnp.float32)
    x2 = jnp.sum(x * x, axis=1, keepdims=True)
    dist = -(x2 - 2.0 * xe + e2_v[...])
    dist_ref[...] = dist
    m = jnp.max(dist, axis=1, keepdims=True)
    iota_k = jax.lax.broadcasted_iota(jnp.int32, dist.shape, 1)
    ind = jnp.min(jnp.where(dist == m, iota_k, _K), axis=1, keepdims=True)
    ind_ref[...] = ind.reshape(_BN // 128, 128)


def _tc_call(xf, embed):
    grid = (_N // _BN,)
    return pl.pallas_call(
        _tc_body,
        grid=grid,
        in_specs=[
            pl.BlockSpec((_BN, _D), lambda i: (i, 0)),
            pl.BlockSpec(memory_space=pltpu.MemorySpace.HBM),
        ],
        out_specs=[
            pl.BlockSpec((_BN // 128, 128), lambda i: (i, 0)),
            pl.BlockSpec((_BN, _K), lambda i: (i, 0)),
        ],
        out_shape=[
            jax.ShapeDtypeStruct((_N // 128, 128), jnp.int32),
            jax.ShapeDtypeStruct((_N, _K), jnp.float32),
        ],
        scratch_shapes=[
            pltpu.VMEM((_K, _D), jnp.float32),
            pltpu.VMEM((_D, _K), jnp.float32),
            pltpu.VMEM((1, _K), jnp.float32),
            pltpu.SemaphoreType.DMA,
        ],
        compiler_params=pltpu.CompilerParams(
            dimension_semantics=("arbitrary",),
        ),
    )(xf, embed)


@functools.partial(
    pl.kernel,
    mesh=plsc.VectorSubcoreMesh(core_axis_name="c", subcore_axis_name="s"),
    out_type=jax.ShapeDtypeStruct((_N, _D), jnp.float32),
    scratch_types=[
        pltpu.VMEM((_BPW,), jnp.int32),
        pltpu.VMEM((_BPW, _D), jnp.float32),
        pltpu.SemaphoreType.DMA,
    ],
)
def _sc_gather(table_hbm, idx_hbm, out_hbm, idx_v, rows_v, sem):
    wid = lax.axis_index("s") * _NC + lax.axis_index("c")
    base = wid * _BPW
    pltpu.sync_copy(idx_hbm.at[pl.ds(base, _BPW)], idx_v)
    pltpu.async_copy(table_hbm.at[idx_v], rows_v, sem).wait()
    pltpu.sync_copy(rows_v, out_hbm.at[pl.ds(base, _BPW)])


def kernel(x, x_len, embed):
    del x_len
    xf = x.reshape(_N, _D)
    ind, dist = _tc_call(xf, embed)
    ind_flat = ind.reshape(_N)
    q = _sc_gather(embed, ind_flat)
    return (q.reshape(_B, _T, _D), ind.reshape(_B, _T), dist.reshape(_B, _T, _K))


# speedup vs baseline: 1.5379x; 1.5379x over previous
import jax
import jax.numpy as jnp
from jax.experimental import pallas as pl
from jax.experimental.pallas import tpu as pltpu

_B, _T, _D = 16, 576, 256
_K = 1024
_N = _B * _T
_BN = 1024


def _body(x_ref, e_hbm, q_ref, ind_ref, dist_ref, e_v, et_v, e2_v, sem):
    i = pl.program_id(0)

    @pl.when(i == 0)
    def _init():
        cp = pltpu.make_async_copy(e_hbm, e_v, sem)
        cp.start()
        cp.wait()
        et = jnp.swapaxes(e_v[...], 0, 1)
        et_v[...] = et
        e2_v[...] = jnp.sum(et * et, axis=0, keepdims=True)

    x = x_ref[...]
    et = et_v[...]
    xe = jax.lax.dot_general(
        x, et, (((1,), (0,)), ((), ())),
        preferred_element_type=jnp.float32)
    x2 = jnp.sum(x * x, axis=1, keepdims=True)
    dist = -(x2 - 2.0 * xe + e2_v[...])
    dist_ref[...] = dist
    m = jnp.max(dist, axis=1, keepdims=True)
    iota_k = jax.lax.broadcasted_iota(jnp.int32, dist.shape, 1)
    ind = jnp.min(jnp.where(dist == m, iota_k, _K), axis=1, keepdims=True)
    ind_ref[...] = ind.reshape(_BN // 128, 128)
    onehot = (iota_k == ind).astype(jnp.float32)
    q_ref[...] = jax.lax.dot_general(
        onehot, et, (((1,), (1,)), ((), ())),
        preferred_element_type=jnp.float32)


def kernel(x, x_len, embed):
    del x_len
    xf = x.reshape(_N, _D)
    grid = (_N // _BN,)
    q, ind, dist = pl.pallas_call(
        _body,
        grid=grid,
        in_specs=[
            pl.BlockSpec((_BN, _D), lambda i: (i, 0)),
            pl.BlockSpec(memory_space=pltpu.MemorySpace.HBM),
        ],
        out_specs=[
            pl.BlockSpec((_BN, _D), lambda i: (i, 0)),
            pl.BlockSpec((_BN // 128, 128), lambda i: (i, 0)),
            pl.BlockSpec((_BN, _K), lambda i: (i, 0)),
        ],
        out_shape=[
            jax.ShapeDtypeStruct((_N, _D), jnp.float32),
            jax.ShapeDtypeStruct((_N // 128, 128), jnp.int32),
            jax.ShapeDtypeStruct((_N, _K), jnp.float32),
        ],
        scratch_shapes=[
            pltpu.VMEM((_K, _D), jnp.float32),
            pltpu.VMEM((_D, _K), jnp.float32),
            pltpu.VMEM((1, _K), jnp.float32),
            pltpu.SemaphoreType.DMA,
        ],
        compiler_params=pltpu.CompilerParams(
            dimension_semantics=("arbitrary",),
        ),
    )(xf, embed)
    return (q.reshape(_B, _T, _D), ind.reshape(_B, _T), dist.reshape(_B, _T, _K))
